# Initial kernel scaffold; baseline (speedup 1.0000x reference)
import functools

import jax
import jax.numpy as jnp
from jax import lax
from jax.experimental import pallas as pl
from jax.experimental.pallas import tpu as pltpu
from jax.experimental.pallas import tpu_sc as plsc

N = 1000
NP = 1024
D = 128
HID = 512
T_STEPS = 10
E_REAL = 17000
EW = 640
E_PAD = EW * 32
NCHUNK = EW // 128
ZROWS = 1024
SENTINEL = 1023 * NP + 1023

E_B = 2000
K_TILES = 16000 // E_B


def _sc_counts(flat_pad):
    mesh = plsc.VectorSubcoreMesh(core_axis_name="c", subcore_axis_name="s")

    @functools.partial(
        pl.kernel,
        out_type=jax.ShapeDtypeStruct((2, NP * NP // 16, 16), jnp.float32),
        mesh=mesh,
        scratch_types=[
            pltpu.VMEM((EW,), jnp.int32),
            pltpu.VMEM((NCHUNK, 128), jnp.int32),
            pltpu.VMEM((EW, 16), jnp.float32),
            pltpu.VMEM((ZROWS, 16), jnp.float32),
            pltpu.VMEM_SHARED((NP * NP // 16, 16), jnp.float32),
        ],
    )
    def sc_kernel(flat_hbm, out_hbm, idx_v, rows_v, vals_v, zbuf_v, c_sh):
        c = lax.axis_index("c")
        s = lax.axis_index("s")
        off = c * (E_PAD // 2) + s * EW
        pltpu.sync_copy(flat_hbm.at[pl.ds(off, EW)], idx_v)

        zero16 = jnp.zeros((16,), jnp.float32)

        def zz(i, _):
            zbuf_v[i] = zero16
            return 0

        lax.fori_loop(0, ZROWS, zz, 0)

        def zv(i, _):
            vals_v[i] = zero16
            return 0

        lax.fori_loop(0, EW, zv, 0)

        iota = lax.iota(jnp.int32, 16)
        ones = jnp.full((16,), 1.0, jnp.float32)
        for g in range(EW // 16):
            fl = idx_v[pl.ds(g * 16, 16)]
            lane = lax.bitwise_and(fl, 15)
            rows16 = lax.shift_right_logical(fl, 4)
            plsc.store_scatter(vals_v, [g * 16 + iota, lane], ones)
            rows_v[g // 8, pl.ds((g % 8) * 16, 16)] = rows16

        rows_per_sub = (NP * NP // 16) // 16
        for k in range(rows_per_sub // ZROWS):
            pltpu.sync_copy(zbuf_v, c_sh.at[pl.ds(s * rows_per_sub + k * ZROWS, ZROWS)])
        plsc.subcore_barrier()

        for ch in range(NCHUNK):
            pltpu.sync_copy(vals_v.at[pl.ds(ch * 128, 128)],
                            c_sh.at[rows_v.at[ch]], add=True)
        plsc.subcore_barrier()

        pltpu.sync_copy(c_sh.at[pl.ds(s * rows_per_sub, rows_per_sub)],
                        out_hbm.at[c, pl.ds(s * rows_per_sub, rows_per_sub)])

    return sc_kernel(flat_pad)


def _tc_body(x_ref, craw_ref, min_ref, mout_ref, wxt_ref, wht_ref, b4_ref,
             wm1t_ref, wm2t_ref, bm_ref, wct_ref, bc_ref, out_ref,
             a_ref, aproj_ref, bproj_ref, xn_ref):
    k = pl.program_id(0)

    @pl.when(k == 0)
    def _prologue():
        cs = craw_ref[0, :N, :N] + craw_ref[1, :N, :N]
        deg = jnp.sum(cs, axis=1)
        dinv = lax.rsqrt(deg)
        a_ref[...] = cs * dinv[:, None] * dinv[None, :]

        def step(t, hc):
            h, cc = hc
            xt = x_ref[t]
            xw = (jnp.dot(xt, wxt_ref[...], preferred_element_type=jnp.float32)
                  + jnp.dot(h, wht_ref[...], preferred_element_type=jnp.float32)
                  + b4_ref[...])
            z = jnp.dot(a_ref[...], xw, preferred_element_type=jnp.float32)
            ig = jax.nn.sigmoid(z[:, 0:128])
            fg = jax.nn.sigmoid(z[:, 128:256])
            og = jax.nn.sigmoid(z[:, 256:384])
            gg = jnp.tanh(z[:, 384:512])
            cc = fg * cc + ig * gg
            h = og * jnp.tanh(cc)
            return h, cc

        h0 = jnp.zeros((N, D), jnp.float32)
        h, _ = lax.fori_loop(0, T_STEPS, step, (h0, h0))
        aproj_ref[...] = jnp.dot(h, wm1t_ref[...], preferred_element_type=jnp.float32)
        bproj_ref[...] = jnp.dot(h, wm2t_ref[...], preferred_element_type=jnp.float32)
        xn_ref[...] = jnp.zeros((N, D), jnp.float32)

    mi = min_ref[...]
    e = jnp.maximum(
        jnp.dot(mi, aproj_ref[...], preferred_element_type=jnp.float32)
        + jnp.dot(mout_ref[...], bproj_ref[...], preferred_element_type=jnp.float32)
        + bm_ref[...], 0.0)
    xn_ref[...] += lax.dot_general(mi, e, (((0,), (0,)), ((), ())),
                                   preferred_element_type=jnp.float32)

    @pl.when(k == K_TILES - 1)
    def _epilogue():
        xw = jnp.dot(xn_ref[...] * (1.0 / N), wct_ref[...],
                     preferred_element_type=jnp.float32)
        out_ref[...] = (jnp.dot(a_ref[...], xw, preferred_element_type=jnp.float32)
                        + bc_ref[...])


def _tc_call(x, craw, m_in, m_out, wxt, wht, b4, wm1t, wm2t, bm2, wct, bc2,
             interpret=False):
    full = lambda a: pl.BlockSpec(a.shape, lambda k, nd=a.ndim: (0,) * nd)
    return pl.pallas_call(
        _tc_body,
        grid=(K_TILES,),
        in_specs=[
            full(x),
            full(craw),
            pl.BlockSpec((E_B, N), lambda k: (k, 0)),
            pl.BlockSpec((E_B, N), lambda k: (k, 0)),
            full(wxt), full(wht), full(b4),
            full(wm1t), full(wm2t), full(bm2), full(wct), full(bc2),
        ],
        out_specs=pl.BlockSpec((N, D), lambda k: (0, 0)),
        out_shape=jax.ShapeDtypeStruct((N, D), jnp.float32),
        scratch_shapes=[
            pltpu.VMEM((N, N), jnp.float32),
            pltpu.VMEM((N, D), jnp.float32),
            pltpu.VMEM((N, D), jnp.float32),
            pltpu.VMEM((N, D), jnp.float32),
        ],
        interpret=interpret,
    )(x, craw, m_in, m_out, wxt, wht, b4, wm1t, wm2t, bm2, wct, bc2)


def kernel(x, edge_index, m_in, m_out, Wi, bi, Wf, bf, Wo, bo, Wg, bg, Wm, bm, Wc, bc):
    n = x.shape[1]
    ar = jnp.arange(n, dtype=edge_index.dtype)
    src = jnp.concatenate([edge_index[0], ar])
    dst = jnp.concatenate([edge_index[1], ar])
    flat = dst.astype(jnp.int32) * NP + src.astype(jnp.int32)
    flat_pad = jnp.concatenate(
        [flat, jnp.full((E_PAD - E_REAL,), SENTINEL, jnp.int32)])

    craw = _sc_counts(flat_pad)
    craw = jnp.reshape(craw, (2, NP, NP))

    wall = jnp.concatenate([Wi, Wf, Wo, Wg], axis=0)
    wxt = wall[:, :D].T
    wht = wall[:, D:].T
    b4 = jnp.concatenate([bi, bf, bo, bg])[None, :]
    wm1t = Wm[:, :D].T
    wm2t = Wm[:, D:].T

    return _tc_call(x, craw, m_in, m_out, wxt, wht, b4, wm1t, wm2t,
                    bm[None, :], Wc.T, bc[None, :])


# speedup vs baseline: 26.9450x; 26.9450x over previous
import functools

import jax
import jax.numpy as jnp
from jax import lax
from jax.experimental import pallas as pl
from jax.experimental.pallas import tpu as pltpu
from jax.experimental.pallas import tpu_sc as plsc

N = 1000
NP = 1024
D = 128
HID = 512
T_STEPS = 10
E_REAL = 17000
EW = 1024
E_PAD = EW * 32
NCHUNK = EW // 128
ZROWS = 1024
SENTINEL = 1023 * NP + 1023

E_B = 2000
K_TILES = 16000 // E_B


ZWORDS = 16384


def _sc_counts(flat2d):
    mesh = plsc.VectorSubcoreMesh(core_axis_name="c", subcore_axis_name="s")

    @functools.partial(
        pl.kernel,
        out_type=jax.ShapeDtypeStruct((2, NP * NP), jnp.float32),
        mesh=mesh,
        scratch_types=[
            pltpu.VMEM((NCHUNK, 128), jnp.int32),
            pltpu.VMEM((NCHUNK, 128), jnp.float32),
            pltpu.VMEM((ZWORDS,), jnp.float32),
            pltpu.VMEM_SHARED((NP * NP,), jnp.float32),
        ],
    )
    def sc_kernel(flat_hbm, out_hbm, idx_v, ones_v, zbuf_v, c_sh):
        c = lax.axis_index("c")
        s = lax.axis_index("s")
        w = s * 2 + c
        pltpu.sync_copy(flat_hbm.at[pl.ds(w * NCHUNK, NCHUNK)], idx_v)

        zero16 = jnp.zeros((16,), jnp.float32)
        ones16 = jnp.full((16,), 1.0, jnp.float32)
        for ch in range(NCHUNK):
            for g in range(8):
                ones_v[ch, pl.ds(g * 16, 16)] = ones16

        def zz(i, _):
            zbuf_v[pl.ds(i * 16, 16)] = zero16
            return 0

        lax.fori_loop(0, ZWORDS // 16, zz, 0)

        words_per_sub = (NP * NP) // 16
        for k in range(words_per_sub // ZWORDS):
            pltpu.sync_copy(zbuf_v,
                            c_sh.at[pl.ds(s * words_per_sub + k * ZWORDS, ZWORDS)])
        plsc.subcore_barrier()

        for ch in range(NCHUNK):
            pltpu.sync_copy(ones_v.at[ch], c_sh.at[idx_v.at[ch]], add=True)
        plsc.subcore_barrier()

        pltpu.sync_copy(c_sh.at[pl.ds(s * words_per_sub, words_per_sub)],
                        out_hbm.at[c, pl.ds(s * words_per_sub, words_per_sub)])

    return sc_kernel(flat2d)


def _tc_body(x_ref, craw_ref, min_ref, mout_ref, wxt_ref, wht_ref, b4_ref,
             wm1t_ref, wm2t_ref, bm_ref, wct_ref, bc_ref, out_ref,
             a_ref, aproj_ref, bproj_ref, xn_ref):
    k = pl.program_id(0)

    @pl.when(k == 0)
    def _prologue():
        cs = craw_ref[0, :N, :N] + craw_ref[1, :N, :N]
        deg = jnp.sum(cs, axis=1)
        dinv = lax.rsqrt(deg)
        a_ref[...] = cs * dinv[:, None] * dinv[None, :]

        def step(t, hc):
            h, cc = hc
            xt = x_ref[t]
            xw = (jnp.dot(xt, wxt_ref[...], preferred_element_type=jnp.float32)
                  + jnp.dot(h, wht_ref[...], preferred_element_type=jnp.float32))
            z = (jnp.dot(a_ref[...], xw, preferred_element_type=jnp.float32)
                 + b4_ref[...])
            ig = jax.nn.sigmoid(z[:, 0:128])
            fg = jax.nn.sigmoid(z[:, 128:256])
            og = jax.nn.sigmoid(z[:, 256:384])
            gg = jnp.tanh(z[:, 384:512])
            cc = fg * cc + ig * gg
            h = og * jnp.tanh(cc)
            return h, cc

        h0 = jnp.zeros((N, D), jnp.float32)
        h, _ = lax.fori_loop(0, T_STEPS, step, (h0, h0))
        aproj_ref[...] = jnp.dot(h, wm1t_ref[...], preferred_element_type=jnp.float32)
        bproj_ref[...] = jnp.dot(h, wm2t_ref[...], preferred_element_type=jnp.float32)
        xn_ref[...] = jnp.zeros((N, D), jnp.float32)

    mi = min_ref[...]
    e = jnp.maximum(
        jnp.dot(mi, aproj_ref[...], preferred_element_type=jnp.float32)
        + jnp.dot(mout_ref[...], bproj_ref[...], preferred_element_type=jnp.float32)
        + bm_ref[...], 0.0)
    xn_ref[...] += lax.dot_general(mi, e, (((0,), (0,)), ((), ())),
                                   preferred_element_type=jnp.float32)

    @pl.when(k == K_TILES - 1)
    def _epilogue():
        xw = jnp.dot(xn_ref[...] * (1.0 / N), wct_ref[...],
                     preferred_element_type=jnp.float32)
        out_ref[...] = (jnp.dot(a_ref[...], xw, preferred_element_type=jnp.float32)
                        + bc_ref[...])


def _tc_call(x, craw, m_in, m_out, wxt, wht, b4, wm1t, wm2t, bm2, wct, bc2,
             interpret=False):
    full = lambda a: pl.BlockSpec(a.shape, lambda k, nd=a.ndim: (0,) * nd)
    return pl.pallas_call(
        _tc_body,
        grid=(K_TILES,),
        in_specs=[
            full(x),
            full(craw),
            pl.BlockSpec((E_B, N), lambda k: (k, 0)),
            pl.BlockSpec((E_B, N), lambda k: (k, 0)),
            full(wxt), full(wht), full(b4),
            full(wm1t), full(wm2t), full(bm2), full(wct), full(bc2),
        ],
        out_specs=pl.BlockSpec((N, D), lambda k: (0, 0)),
        out_shape=jax.ShapeDtypeStruct((N, D), jnp.float32),
        scratch_shapes=[
            pltpu.VMEM((N, N), jnp.float32),
            pltpu.VMEM((N, D), jnp.float32),
            pltpu.VMEM((N, D), jnp.float32),
            pltpu.VMEM((N, D), jnp.float32),
        ],
        interpret=interpret,
    )(x, craw, m_in, m_out, wxt, wht, b4, wm1t, wm2t, bm2, wct, bc2)


def kernel(x, edge_index, m_in, m_out, Wi, bi, Wf, bf, Wo, bo, Wg, bg, Wm, bm, Wc, bc):
    n = x.shape[1]
    ar = jnp.arange(n, dtype=edge_index.dtype)
    src = jnp.concatenate([edge_index[0], ar])
    dst = jnp.concatenate([edge_index[1], ar])
    flat = dst.astype(jnp.int32) * NP + src.astype(jnp.int32)
    flat_pad = jnp.concatenate(
        [flat, jnp.full((E_PAD - E_REAL,), SENTINEL, jnp.int32)])

    craw = _sc_counts(jnp.reshape(flat_pad, (E_PAD // 128, 128)))
    craw = jnp.reshape(craw, (2, NP, NP))

    wall = jnp.concatenate([Wi, Wf, Wo, Wg], axis=0)
    wxt = wall[:, :D].T
    wht = wall[:, D:].T
    b4 = jnp.concatenate([bi, bf, bo, bg])[None, :]
    wm1t = Wm[:, :D].T
    wm2t = Wm[:, D:].T

    return _tc_call(x, craw, m_in, m_out, wxt, wht, b4, wm1t, wm2t,
                    bm[None, :], Wc.T, bc[None, :])


# speedup vs baseline: 29.4481x; 1.0929x over previous
import functools

import jax
import jax.numpy as jnp
from jax import lax
from jax.experimental import pallas as pl
from jax.experimental.pallas import tpu as pltpu
from jax.experimental.pallas import tpu_sc as plsc

N = 1000
NP = 1024
D = 128
HID = 512
T_STEPS = 10
E_REAL = 17000
EW = 1024
E_PAD = EW * 32
NCHUNK = EW // 128
ZROWS = 1024
SENTINEL = 1023 * NP + 1023

E_B = 2000
K_TILES = 16000 // E_B


ZWORDS = 16384


def _sc_counts(flat2d):
    mesh = plsc.VectorSubcoreMesh(core_axis_name="c", subcore_axis_name="s")

    @functools.partial(
        pl.kernel,
        out_type=jax.ShapeDtypeStruct((2, NP, NP), jnp.float32),
        mesh=mesh,
        scratch_types=[
            pltpu.VMEM((NCHUNK, 128), jnp.int32),
            pltpu.VMEM((NCHUNK, 128), jnp.float32),
            pltpu.VMEM((ZWORDS,), jnp.float32),
            pltpu.VMEM_SHARED((NP * NP,), jnp.float32),
        ],
    )
    def sc_kernel(flat_hbm, out_hbm, idx_v, ones_v, zbuf_v, c_sh):
        c = lax.axis_index("c")
        s = lax.axis_index("s")
        w = s * 2 + c
        pltpu.sync_copy(flat_hbm.at[pl.ds(w * NCHUNK, NCHUNK)], idx_v)

        zero16 = jnp.zeros((16,), jnp.float32)
        ones16 = jnp.full((16,), 1.0, jnp.float32)
        for ch in range(NCHUNK):
            for g in range(8):
                ones_v[ch, pl.ds(g * 16, 16)] = ones16

        def zz(i, _):
            zbuf_v[pl.ds(i * 16, 16)] = zero16
            return 0

        lax.fori_loop(0, ZWORDS // 16, zz, 0)

        words_per_sub = (NP * NP) // 16
        for k in range(words_per_sub // ZWORDS):
            pltpu.sync_copy(zbuf_v,
                            c_sh.at[pl.ds(s * words_per_sub + k * ZWORDS, ZWORDS)])
        plsc.subcore_barrier()

        for ch in range(NCHUNK):
            pltpu.sync_copy(ones_v.at[ch], c_sh.at[idx_v.at[ch]], add=True)
        plsc.subcore_barrier()

        for r in range(words_per_sub // NP):
            pltpu.sync_copy(c_sh.at[pl.ds(s * words_per_sub + r * NP, NP)],
                            out_hbm.at[c, s * (words_per_sub // NP) + r])

    return sc_kernel(flat2d)


def _dot16(a, b):
    return jnp.dot(a.astype(jnp.bfloat16), b.astype(jnp.bfloat16),
                   preferred_element_type=jnp.float32)


def _tc_body(x_ref, craw_ref, min_ref, mout_ref, wxt_ref, wht_ref, b4_ref,
             wm1t_ref, wm2t_ref, bm_ref, wct_ref, bc_ref, out_ref,
             a16_ref, aproj_ref, bproj_ref, xn_ref):
    k = pl.program_id(0)

    @pl.when(k == 0)
    def _prologue():
        cs = craw_ref[0, :N, :N] + craw_ref[1, :N, :N]
        deg = jnp.sum(cs, axis=1)
        dinv = lax.rsqrt(deg)
        a16_ref[...] = (cs * dinv[:, None] * dinv[None, :]).astype(jnp.bfloat16)

        wx16 = wxt_ref[...].astype(jnp.bfloat16)
        wh16 = wht_ref[...].astype(jnp.bfloat16)

        def step(t, hc):
            h, cc = hc
            xt = x_ref[t].astype(jnp.bfloat16)
            xw = (jnp.dot(xt, wx16, preferred_element_type=jnp.float32)
                  + jnp.dot(h.astype(jnp.bfloat16), wh16,
                            preferred_element_type=jnp.float32))
            z = (jnp.dot(a16_ref[...], xw.astype(jnp.bfloat16),
                         preferred_element_type=jnp.float32)
                 + b4_ref[...])
            ig = jax.nn.sigmoid(z[:, 0:128])
            fg = jax.nn.sigmoid(z[:, 128:256])
            og = jax.nn.sigmoid(z[:, 256:384])
            gg = jnp.tanh(z[:, 384:512])
            cc = fg * cc + ig * gg
            h = og * jnp.tanh(cc)
            return h, cc

        h0 = jnp.zeros((N, D), jnp.float32)
        h, _ = lax.fori_loop(0, T_STEPS, step, (h0, h0))
        aproj_ref[...] = _dot16(h, wm1t_ref[...]).astype(jnp.bfloat16)
        bproj_ref[...] = _dot16(h, wm2t_ref[...]).astype(jnp.bfloat16)
        xn_ref[...] = jnp.zeros((N, D), jnp.float32)

    mi16 = min_ref[...].astype(jnp.bfloat16)
    e = jnp.maximum(
        jnp.dot(mi16, aproj_ref[...], preferred_element_type=jnp.float32)
        + jnp.dot(mout_ref[...].astype(jnp.bfloat16), bproj_ref[...],
                  preferred_element_type=jnp.float32)
        + bm_ref[...], 0.0)
    xn_ref[...] += lax.dot_general(mi16, e.astype(jnp.bfloat16),
                                   (((0,), (0,)), ((), ())),
                                   preferred_element_type=jnp.float32)

    @pl.when(k == K_TILES - 1)
    def _epilogue():
        xw = _dot16(xn_ref[...] * (1.0 / N), wct_ref[...])
        out_ref[...] = (jnp.dot(a16_ref[...], xw.astype(jnp.bfloat16),
                                preferred_element_type=jnp.float32)
                        + bc_ref[...])


def _tc_call(x, craw, m_in, m_out, wxt, wht, b4, wm1t, wm2t, bm2, wct, bc2,
             interpret=False):
    full = lambda a: pl.BlockSpec(a.shape, lambda k, nd=a.ndim: (0,) * nd)
    return pl.pallas_call(
        _tc_body,
        grid=(K_TILES,),
        in_specs=[
            full(x),
            full(craw),
            pl.BlockSpec((E_B, N), lambda k: (k, 0)),
            pl.BlockSpec((E_B, N), lambda k: (k, 0)),
            full(wxt), full(wht), full(b4),
            full(wm1t), full(wm2t), full(bm2), full(wct), full(bc2),
        ],
        out_specs=pl.BlockSpec((N, D), lambda k: (0, 0)),
        out_shape=jax.ShapeDtypeStruct((N, D), jnp.float32),
        scratch_shapes=[
            pltpu.VMEM((N, N), jnp.bfloat16),
            pltpu.VMEM((N, D), jnp.bfloat16),
            pltpu.VMEM((N, D), jnp.bfloat16),
            pltpu.VMEM((N, D), jnp.float32),
        ],
        interpret=interpret,
    )(x, craw, m_in, m_out, wxt, wht, b4, wm1t, wm2t, bm2, wct, bc2)


def kernel(x, edge_index, m_in, m_out, Wi, bi, Wf, bf, Wo, bo, Wg, bg, Wm, bm, Wc, bc):
    n = x.shape[1]
    ar = jnp.arange(n, dtype=edge_index.dtype)
    src = jnp.concatenate([edge_index[0], ar])
    dst = jnp.concatenate([edge_index[1], ar])
    flat = dst.astype(jnp.int32) * NP + src.astype(jnp.int32)
    flat_pad = jnp.concatenate(
        [flat, jnp.full((E_PAD - E_REAL,), SENTINEL, jnp.int32)])

    craw = _sc_counts(jnp.reshape(flat_pad, (E_PAD // 128, 128)))

    wall = jnp.concatenate([Wi, Wf, Wo, Wg], axis=0)
    wxt = wall[:, :D].T
    wht = wall[:, D:].T
    b4 = jnp.concatenate([bi, bf, bo, bg])[None, :]
    wm1t = Wm[:, :D].T
    wm2t = Wm[:, D:].T

    return _tc_call(x, craw, m_in, m_out, wxt, wht, b4, wm1t, wm2t,
                    bm[None, :], Wc.T, bc[None, :])
